# baseline (speedup 1.0000x reference)
import functools

import jax
import jax.numpy as jnp
from jax import lax
from jax.experimental import pallas as pl
from jax.experimental.pallas import tpu as pltpu
from jax.experimental.pallas import tpu_sc as plsc

B, N, D = 16, 4096, 256
NC, NS, L = 2, 16, 16
NJ = D // L
RH = N // 2
CH = 128
NCHUNK = RH // CH

_mesh = plsc.VectorSubcoreMesh(core_axis_name="c", subcore_axis_name="s")


@functools.partial(
    pl.kernel,
    mesh=_mesh,
    out_type=jax.ShapeDtypeStruct((2, B, 4, D), jnp.float32),
    scratch_types=[
        pltpu.VMEM((CH, D), jnp.float32),
        pltpu.VMEM((4, D), jnp.float32),
    ],
)
def _pool_partials(x_hbm, part_hbm, buf, acc):
    c = lax.axis_index("c")
    s = lax.axis_index("s")
    b = c * (B // NC) + s // 2
    rh = s % 2
    r0 = rh * RH

    for j in range(NJ):
        sl = pl.ds(j * L, L)
        acc[0, sl] = jnp.full((L,), -jnp.inf, jnp.float32)
        acc[1, sl] = jnp.full((L,), jnp.inf, jnp.float32)
        acc[2, sl] = jnp.zeros((L,), jnp.float32)
        acc[3, sl] = jnp.zeros((L,), jnp.float32)

    def chunk_body(k, carry):
        pltpu.sync_copy(x_hbm.at[b, pl.ds(r0 + k * CH, CH), :], buf)
        for j in range(NJ):
            sl = pl.ds(j * L, L)

            def row_body(r, t):
                mx, mn, sm, ss = t
                v = buf[r, sl]
                return (jnp.maximum(mx, v), jnp.minimum(mn, v),
                        sm + v, ss + v * v)

            mx, mn, sm, ss = lax.fori_loop(
                0, CH, row_body,
                (acc[0, sl], acc[1, sl], acc[2, sl], acc[3, sl]))
            acc[0, sl] = mx
            acc[1, sl] = mn
            acc[2, sl] = sm
            acc[3, sl] = ss
        return carry

    lax.fori_loop(0, NCHUNK, chunk_body, 0)
    pltpu.sync_copy(acc, part_hbm.at[rh, b])


def _finalize_body(p_ref, o_ref):
    a = p_ref[0]
    z = p_ref[1]
    mx = jnp.maximum(a[:, 0], z[:, 0])
    mn = jnp.minimum(a[:, 1], z[:, 1])
    sm = a[:, 2] + z[:, 2]
    ss = a[:, 3] + z[:, 3]
    mean = sm * jnp.float32(1.0 / N)
    var = jnp.maximum((ss - sm * mean) * jnp.float32(1.0 / (N - 1)), 0.0)
    std = jnp.sqrt(var)
    o_ref[...] = jnp.stack([mx, mn, mean, std], axis=1)


_finalize = pl.pallas_call(
    _finalize_body,
    out_shape=jax.ShapeDtypeStruct((B, 4, D), jnp.float32),
)


def kernel(x):
    return _finalize(_pool_partials(x))


# speedup vs baseline: 2.7788x; 2.7788x over previous
import functools

import jax
import jax.numpy as jnp
from jax import lax
from jax.experimental import pallas as pl
from jax.experimental.pallas import tpu as pltpu
from jax.experimental.pallas import tpu_sc as plsc

B, N, D = 16, 4096, 256
NC, NS, L = 2, 16, 16
NJ = D // L
RH = N // 2
CH = 128
NCHUNK = RH // CH
RSTEP = 8

_mesh = plsc.VectorSubcoreMesh(core_axis_name="c", subcore_axis_name="s")


def _tree(op, xs):
    while len(xs) > 1:
        xs = [op(xs[i], xs[i + 1]) for i in range(0, len(xs) - 1, 2)] + (
            [xs[-1]] if len(xs) % 2 else [])
    return xs[0]


@functools.partial(
    pl.kernel,
    mesh=_mesh,
    out_type=jax.ShapeDtypeStruct((2, B, 4, D), jnp.float32),
    scratch_types=[
        pltpu.VMEM((CH, D), jnp.float32),
        pltpu.VMEM((CH, D), jnp.float32),
        pltpu.VMEM((4, D), jnp.float32),
        pltpu.SemaphoreType.DMA,
        pltpu.SemaphoreType.DMA,
    ],
)
def _pool_partials(x_hbm, part_hbm, buf0, buf1, acc, sem0, sem1):
    c = lax.axis_index("c")
    s = lax.axis_index("s")
    b = c * (B // NC) + s // 2
    rh = s % 2
    r0 = rh * RH

    for j in range(NJ):
        sl = pl.ds(j * L, L)
        acc[0, sl] = jnp.full((L,), -jnp.inf, jnp.float32)
        acc[1, sl] = jnp.full((L,), jnp.inf, jnp.float32)
        acc[2, sl] = jnp.zeros((L,), jnp.float32)
        acc[3, sl] = jnp.zeros((L,), jnp.float32)

    def copy(k, buf, sem):
        return pltpu.make_async_copy(
            x_hbm.at[b, pl.ds(r0 + k * CH, CH), :], buf, sem)

    copy(0, buf0, sem0).start()
    copy(1, buf1, sem1).start()

    def process(buf):
        def jbody(j, carry):
            sl = pl.ds(j * L, L)

            @plsc.parallel_loop(
                0, CH, step=RSTEP,
                carry=(acc[0, sl], acc[1, sl], acc[2, sl], acc[3, sl]))
            def rbody(r, t):
                mx, mn, sm, ss = t
                v = [buf[r + u, sl] for u in range(RSTEP)]
                mx = jnp.maximum(mx, _tree(jnp.maximum, v))
                mn = jnp.minimum(mn, _tree(jnp.minimum, v))
                sm = sm + _tree(lax.add, v)
                ss = ss + _tree(lax.add, [vi * vi for vi in v])
                return (mx, mn, sm, ss)

            mx, mn, sm, ss = rbody
            acc[0, sl] = mx
            acc[1, sl] = mn
            acc[2, sl] = sm
            acc[3, sl] = ss
            return carry

        lax.fori_loop(0, NJ, jbody, 0)

    def outer(i, carry):
        kk = i * 2
        copy(kk, buf0, sem0).wait()
        process(buf0)

        @pl.when(kk + 2 < NCHUNK)
        def _():
            copy(kk + 2, buf0, sem0).start()

        copy(kk + 1, buf1, sem1).wait()
        process(buf1)

        @pl.when(kk + 3 < NCHUNK)
        def _():
            copy(kk + 3, buf1, sem1).start()

        return carry

    lax.fori_loop(0, NCHUNK // 2, outer, 0)
    pltpu.sync_copy(acc, part_hbm.at[rh, b])


def _finalize_body(p_ref, o_ref):
    a = p_ref[0]
    z = p_ref[1]
    mx = jnp.maximum(a[:, 0], z[:, 0])
    mn = jnp.minimum(a[:, 1], z[:, 1])
    sm = a[:, 2] + z[:, 2]
    ss = a[:, 3] + z[:, 3]
    mean = sm * jnp.float32(1.0 / N)
    var = jnp.maximum((ss - sm * mean) * jnp.float32(1.0 / (N - 1)), 0.0)
    std = jnp.sqrt(var)
    o_ref[...] = jnp.stack([mx, mn, mean, std], axis=1)


_finalize = pl.pallas_call(
    _finalize_body,
    out_shape=jax.ShapeDtypeStruct((B, 4, D), jnp.float32),
)


def kernel(x):
    return _finalize(_pool_partials(x))
